# baseline (speedup 1.0000x reference)
import jax
import jax.numpy as jnp
from jax.experimental import pallas as pl
from jax.experimental.pallas import tpu as pltpu

_ALPHA = 0.3
_ROWS = 2048
_CH = 32
_UNROLL = 8


def _make_body(n_steps, inv_n, lanes):
    def _loss_body(x_ref, y_ref, o_ref, acc_ref):
        i = pl.program_id(0)

        @pl.when(i == 0)
        def _():
            acc_ref[...] = jnp.zeros_like(acc_ref)

        def chunk(k, acc):
            parts = []
            for u in range(_UNROLL):
                xs = x_ref[pl.ds((k * _UNROLL + u) * _CH, _CH), :]
                ys = y_ref[pl.ds((k * _UNROLL + u) * _CH, _CH), :]
                d = ys - xs
                v = jnp.maximum((1.0 - _ALPHA) * d, -_ALPHA * d)
                parts.append(v.reshape(_CH // 8, 8, lanes).sum(axis=0))
            p = parts[0]
            for q in parts[1:]:
                p = p + q
            return acc + p

        acc_ref[...] += jax.lax.fori_loop(
            0, _ROWS // (_CH * _UNROLL), chunk,
            jnp.zeros((8, lanes), jnp.float32)
        )

        @pl.when(i == n_steps - 1)
        def _():
            o_ref[0, 0] = jnp.sum(acc_ref[...]) * inv_n

    return _loss_body


def kernel(x, y):
    n = x.size
    w = x.shape[-1]
    x2 = x.reshape(-1, w)
    y2 = y.reshape(-1, w)
    grid = x2.shape[0] // _ROWS
    out = pl.pallas_call(
        _make_body(grid, 1.0 / n, w),
        grid=(grid,),
        in_specs=[
            pl.BlockSpec((_ROWS, w), lambda i: (i, 0)),
            pl.BlockSpec((_ROWS, w), lambda i: (i, 0)),
        ],
        out_specs=pl.BlockSpec(memory_space=pltpu.SMEM),
        out_shape=jax.ShapeDtypeStruct((1, 1), jnp.float32),
        scratch_shapes=[pltpu.VMEM((8, w), jnp.float32)],
        compiler_params=pltpu.CompilerParams(
            dimension_semantics=("arbitrary",),
        ),
    )(x2, y2)
    return out.reshape(())


# speedup vs baseline: 1.0060x; 1.0060x over previous
import jax
import jax.numpy as jnp
from jax.experimental import pallas as pl
from jax.experimental.pallas import tpu as pltpu

_ALPHA = 0.3
_CHR = 1024
_NBUF = 4
_CH = 32
_UNROLL = 4


def _make_body(inv_n, lanes, total_rows):
    n_chunks = total_rows // _CHR

    def body(x_hbm, y_hbm, o_ref, xb, yb, acc_ref, sem):
        def start(k):
            slot = k % _NBUF
            pltpu.make_async_copy(
                x_hbm.at[pl.ds(k * _CHR, _CHR), :], xb.at[slot],
                sem.at[0, slot]).start()
            pltpu.make_async_copy(
                y_hbm.at[pl.ds(k * _CHR, _CHR), :], yb.at[slot],
                sem.at[1, slot]).start()

        for k in range(_NBUF):
            start(k)

        acc_ref[...] = jnp.zeros_like(acc_ref)

        for k in range(n_chunks):
            slot = k % _NBUF
            pltpu.make_async_copy(xb.at[slot], xb.at[slot],
                                  sem.at[0, slot]).wait()
            pltpu.make_async_copy(yb.at[slot], yb.at[slot],
                                  sem.at[1, slot]).wait()

            def chunk(j, acc, slot=slot):
                parts = []
                for u in range(_UNROLL):
                    r = (j * _UNROLL + u) * _CH
                    xs = xb[slot, pl.ds(r, _CH), :]
                    ys = yb[slot, pl.ds(r, _CH), :]
                    d = ys - xs
                    v = jnp.maximum((1.0 - _ALPHA) * d, -_ALPHA * d)
                    parts.append(v.reshape(_CH // 8, 8, lanes).sum(axis=0))
                p = parts[0]
                for q in parts[1:]:
                    p = p + q
                return acc + p

            acc_ref[...] += jax.lax.fori_loop(
                0, _CHR // (_CH * _UNROLL), chunk,
                jnp.zeros((8, lanes), jnp.float32))

            if k + _NBUF < n_chunks:
                start(k + _NBUF)

        o_ref[0, 0] = jnp.sum(acc_ref[...]) * inv_n

    return body


def kernel(x, y):
    n = x.size
    w = x.shape[-1]
    x2 = x.reshape(-1, w)
    y2 = y.reshape(-1, w)
    out = pl.pallas_call(
        _make_body(1.0 / n, w, x2.shape[0]),
        in_specs=[
            pl.BlockSpec(memory_space=pl.ANY),
            pl.BlockSpec(memory_space=pl.ANY),
        ],
        out_specs=pl.BlockSpec(memory_space=pltpu.SMEM),
        out_shape=jax.ShapeDtypeStruct((1, 1), jnp.float32),
        scratch_shapes=[
            pltpu.VMEM((_NBUF, _CHR, w), jnp.float32),
            pltpu.VMEM((_NBUF, _CHR, w), jnp.float32),
            pltpu.VMEM((8, w), jnp.float32),
            pltpu.SemaphoreType.DMA((2, _NBUF)),
        ],
    )(x2, y2)
    return out.reshape(())


# speedup vs baseline: 1.0092x; 1.0031x over previous
import jax
import jax.numpy as jnp
from jax.experimental import pallas as pl
from jax.experimental.pallas import tpu as pltpu

_ALPHA = 0.3
_CHR = 512
_NBUF = 8
_CH = 32
_UNROLL = 4


def _make_body(inv_n, lanes, total_rows):
    n_chunks = total_rows // _CHR

    def body(x_hbm, y_hbm, o_ref, xb, yb, acc_ref, sem):
        def start(k):
            slot = k % _NBUF
            pltpu.make_async_copy(
                x_hbm.at[pl.ds(k * _CHR, _CHR), :], xb.at[slot],
                sem.at[0, slot]).start()
            pltpu.make_async_copy(
                y_hbm.at[pl.ds(k * _CHR, _CHR), :], yb.at[slot],
                sem.at[1, slot]).start()

        for k in range(_NBUF):
            start(k)

        acc_ref[...] = jnp.zeros_like(acc_ref)

        for k in range(n_chunks):
            slot = k % _NBUF
            pltpu.make_async_copy(xb.at[slot], xb.at[slot],
                                  sem.at[0, slot]).wait()
            pltpu.make_async_copy(yb.at[slot], yb.at[slot],
                                  sem.at[1, slot]).wait()

            def chunk(j, acc, slot=slot):
                parts = []
                for u in range(_UNROLL):
                    r = (j * _UNROLL + u) * _CH
                    xs = xb[slot, pl.ds(r, _CH), :]
                    ys = yb[slot, pl.ds(r, _CH), :]
                    d = ys - xs
                    v = jnp.maximum((1.0 - _ALPHA) * d, -_ALPHA * d)
                    parts.append(v.reshape(_CH // 8, 8, lanes).sum(axis=0))
                p = parts[0]
                for q in parts[1:]:
                    p = p + q
                return acc + p

            acc_ref[...] += jax.lax.fori_loop(
                0, _CHR // (_CH * _UNROLL), chunk,
                jnp.zeros((8, lanes), jnp.float32))

            if k + _NBUF < n_chunks:
                start(k + _NBUF)

        o_ref[0, 0] = jnp.sum(acc_ref[...]) * inv_n

    return body


def kernel(x, y):
    n = x.size
    w = x.shape[-1]
    x2 = x.reshape(-1, w)
    y2 = y.reshape(-1, w)
    out = pl.pallas_call(
        _make_body(1.0 / n, w, x2.shape[0]),
        in_specs=[
            pl.BlockSpec(memory_space=pl.ANY),
            pl.BlockSpec(memory_space=pl.ANY),
        ],
        out_specs=pl.BlockSpec(memory_space=pltpu.SMEM),
        out_shape=jax.ShapeDtypeStruct((1, 1), jnp.float32),
        scratch_shapes=[
            pltpu.VMEM((_NBUF, _CHR, w), jnp.float32),
            pltpu.VMEM((_NBUF, _CHR, w), jnp.float32),
            pltpu.VMEM((8, w), jnp.float32),
            pltpu.SemaphoreType.DMA((2, _NBUF)),
        ],
    )(x2, y2)
    return out.reshape(())


# speedup vs baseline: 1.0103x; 1.0011x over previous
import jax
import jax.numpy as jnp
from jax.experimental import pallas as pl
from jax.experimental.pallas import tpu as pltpu

_ALPHA = 0.3
_CHR = 256
_NBUF = 12
_CH = 32
_UNROLL = 4


def _make_body(inv_n, lanes, total_rows):
    n_chunks = total_rows // _CHR

    def body(x_hbm, y_hbm, o_ref, xb, yb, acc_ref, sem):
        def start(k):
            slot = k % _NBUF
            pltpu.make_async_copy(
                x_hbm.at[pl.ds(k * _CHR, _CHR), :], xb.at[slot],
                sem.at[0, slot]).start()
            pltpu.make_async_copy(
                y_hbm.at[pl.ds(k * _CHR, _CHR), :], yb.at[slot],
                sem.at[1, slot]).start()

        for k in range(_NBUF):
            start(k)

        acc_ref[...] = jnp.zeros_like(acc_ref)

        for k in range(n_chunks):
            slot = k % _NBUF
            pltpu.make_async_copy(xb.at[slot], xb.at[slot],
                                  sem.at[0, slot]).wait()
            pltpu.make_async_copy(yb.at[slot], yb.at[slot],
                                  sem.at[1, slot]).wait()

            def chunk(j, acc, slot=slot):
                parts = []
                for u in range(_UNROLL):
                    r = (j * _UNROLL + u) * _CH
                    xs = xb[slot, pl.ds(r, _CH), :]
                    ys = yb[slot, pl.ds(r, _CH), :]
                    d = ys - xs
                    v = jnp.maximum((1.0 - _ALPHA) * d, -_ALPHA * d)
                    parts.append(v.reshape(_CH // 8, 8, lanes).sum(axis=0))
                p = parts[0]
                for q in parts[1:]:
                    p = p + q
                return acc + p

            acc_ref[...] += jax.lax.fori_loop(
                0, _CHR // (_CH * _UNROLL), chunk,
                jnp.zeros((8, lanes), jnp.float32))

            if k + _NBUF < n_chunks:
                start(k + _NBUF)

        o_ref[0, 0] = jnp.sum(acc_ref[...]) * inv_n

    return body


def kernel(x, y):
    n = x.size
    w = x.shape[-1]
    x2 = x.reshape(-1, w)
    y2 = y.reshape(-1, w)
    out = pl.pallas_call(
        _make_body(1.0 / n, w, x2.shape[0]),
        in_specs=[
            pl.BlockSpec(memory_space=pl.ANY),
            pl.BlockSpec(memory_space=pl.ANY),
        ],
        out_specs=pl.BlockSpec(memory_space=pltpu.SMEM),
        out_shape=jax.ShapeDtypeStruct((1, 1), jnp.float32),
        scratch_shapes=[
            pltpu.VMEM((_NBUF, _CHR, w), jnp.float32),
            pltpu.VMEM((_NBUF, _CHR, w), jnp.float32),
            pltpu.VMEM((8, w), jnp.float32),
            pltpu.SemaphoreType.DMA((2, _NBUF)),
        ],
    )(x2, y2)
    return out.reshape(())


# speedup vs baseline: 1.0139x; 1.0036x over previous
import jax
import jax.numpy as jnp
from jax.experimental import pallas as pl
from jax.experimental.pallas import tpu as pltpu

_ALPHA = 0.3
_CHR = 128
_NBUF = 16
_CH = 32
_UNROLL = 4


def _make_body(inv_n, lanes, total_rows):
    n_chunks = total_rows // _CHR

    def body(x_hbm, y_hbm, o_ref, xb, yb, acc_ref, sem):
        def start(k):
            slot = k % _NBUF
            pltpu.make_async_copy(
                x_hbm.at[pl.ds(k * _CHR, _CHR), :], xb.at[slot],
                sem.at[0, slot]).start()
            pltpu.make_async_copy(
                y_hbm.at[pl.ds(k * _CHR, _CHR), :], yb.at[slot],
                sem.at[1, slot]).start()

        for k in range(_NBUF):
            start(k)

        acc_ref[...] = jnp.zeros_like(acc_ref)

        for k in range(n_chunks):
            slot = k % _NBUF
            pltpu.make_async_copy(xb.at[slot], xb.at[slot],
                                  sem.at[0, slot]).wait()
            pltpu.make_async_copy(yb.at[slot], yb.at[slot],
                                  sem.at[1, slot]).wait()

            def chunk(j, acc, slot=slot):
                parts = []
                for u in range(_UNROLL):
                    r = (j * _UNROLL + u) * _CH
                    xs = xb[slot, pl.ds(r, _CH), :]
                    ys = yb[slot, pl.ds(r, _CH), :]
                    d = ys - xs
                    v = jnp.maximum((1.0 - _ALPHA) * d, -_ALPHA * d)
                    parts.append(v.reshape(_CH // 8, 8, lanes).sum(axis=0))
                p = parts[0]
                for q in parts[1:]:
                    p = p + q
                return acc + p

            acc_ref[...] += jax.lax.fori_loop(
                0, _CHR // (_CH * _UNROLL), chunk,
                jnp.zeros((8, lanes), jnp.float32))

            if k + _NBUF < n_chunks:
                start(k + _NBUF)

        o_ref[0, 0] = jnp.sum(acc_ref[...]) * inv_n

    return body


def kernel(x, y):
    n = x.size
    w = x.shape[-1]
    x2 = x.reshape(-1, w)
    y2 = y.reshape(-1, w)
    out = pl.pallas_call(
        _make_body(1.0 / n, w, x2.shape[0]),
        in_specs=[
            pl.BlockSpec(memory_space=pl.ANY),
            pl.BlockSpec(memory_space=pl.ANY),
        ],
        out_specs=pl.BlockSpec(memory_space=pltpu.SMEM),
        out_shape=jax.ShapeDtypeStruct((1, 1), jnp.float32),
        scratch_shapes=[
            pltpu.VMEM((_NBUF, _CHR, w), jnp.float32),
            pltpu.VMEM((_NBUF, _CHR, w), jnp.float32),
            pltpu.VMEM((8, w), jnp.float32),
            pltpu.SemaphoreType.DMA((2, _NBUF)),
        ],
    )(x2, y2)
    return out.reshape(())
